# baseline (speedup 1.0000x reference)
import functools

import jax
import jax.numpy as jnp
from jax import lax
from jax.experimental import pallas as pl
from jax.experimental.pallas import tpu as pltpu
from jax.experimental.pallas import tpu_sc as plsc

N = 10000
NPAD = 10240
E = 320000
D = 128
H = 128
C = 10
G = 64

NC = 2
NS = 16
NW = NC * NS
EPW = E // NW
CH = 80
NCH = EPW // CH
NB = 4
NGR = 32
EPWP = NGR * NB * CH
ACCR = NPAD + 8
RPS = NPAD // NS

_mesh = plsc.VectorSubcoreMesh(core_axis_name="c", subcore_axis_name="s")


@functools.partial(
    pl.kernel,
    out_type=jax.ShapeDtypeStruct((NC, NPAD, 16), jnp.float32),
    mesh=_mesh,
    scratch_types=[
        pltpu.VMEM((NCH, CH), jnp.int32),
        pltpu.VMEM((CH, 16), jnp.float32),
        pltpu.VMEM((16, 16), jnp.float32),
        pltpu.VMEM_SHARED((NPAD, 16), jnp.float32),
    ],
)
def _sc_degree(dst_hbm, out_hbm, idx_d, ones_b, zb, acc_sh):
    c = lax.axis_index("c")
    s = lax.axis_index("s")
    w = s * NC + c
    pltpu.sync_copy(dst_hbm.at[w], idx_d)

    one16 = jnp.full((16,), 1.0, dtype=jnp.float32)
    zero16 = jnp.zeros((16,), dtype=jnp.float32)

    def fill_ones(i, carry):
        ones_b[i, pl.ds(0, 16)] = one16
        return carry

    lax.fori_loop(0, CH, fill_ones, 0)

    def fill_zero(i, carry):
        zb[i, pl.ds(0, 16)] = zero16
        return carry

    lax.fori_loop(0, 16, fill_zero, 0)

    def zero_acc(i, carry):
        pltpu.sync_copy(zb, acc_sh.at[pl.ds(s * RPS + i * 16, 16)])
        return carry

    lax.fori_loop(0, RPS // 16, zero_acc, 0)
    plsc.subcore_barrier()

    def chunk(j, carry):
        pltpu.sync_copy(ones_b, acc_sh.at[idx_d.at[j]], add=True)
        return carry

    lax.fori_loop(0, NCH, chunk, 0)
    plsc.subcore_barrier()
    pltpu.sync_copy(acc_sh.at[pl.ds(s * RPS, RPS)],
                    out_hbm.at[c, pl.ds(s * RPS, RPS)])


@functools.partial(
    pl.kernel,
    out_type=jax.ShapeDtypeStruct((NC, NPAD, H), jnp.float32),
    mesh=_mesh,
    scratch_types=[
        pltpu.VMEM((2, 2 * NB, CH), jnp.int32),
        pltpu.VMEM((NB * CH, H), jnp.float32),
        pltpu.SemaphoreType.DMA((NB,)),
        pltpu.SemaphoreType.DMA((2,)),
        pltpu.VMEM_SHARED((ACCR, H), jnp.float32),
    ],
    compiler_params=pltpu.CompilerParams(use_tc_tiling_on_sc=False),
)
def _sc_aggregate(y_hbm, grp_hbm, out_hbm, idxg, rowsb, gsems, isems, acc_sh):
    c = lax.axis_index("c")
    s = lax.axis_index("s")
    w = s * NC + c

    zero16 = jnp.zeros((16,), dtype=jnp.float32)

    def fill_zero(i, carry):
        rowsb[i // 8, pl.ds((i % 8) * 16, 16)] = zero16
        return carry

    lax.fori_loop(0, CH * (H // 16), fill_zero, 0)

    def zero_acc(i, carry):
        pltpu.sync_copy(rowsb.at[pl.ds(0, 80)],
                        acc_sh.at[pl.ds(s * RPS + i * 80, 80)])
        return carry

    lax.fori_loop(0, RPS // 80, zero_acc, 0)

    pltpu.sync_copy(grp_hbm.at[w, 0], idxg.at[0])
    for b in range(NB):
        pltpu.async_copy(y_hbm.at[idxg.at[0, b]],
                         rowsb.at[pl.ds(b * CH, CH)], gsems.at[b])
    pltpu.async_copy(grp_hbm.at[w, 1], idxg.at[1], isems.at[1])
    plsc.subcore_barrier()

    def grouppair(g, carry):
        for par in range(2):
            gg = g * 2 + par
            cur, nxt = par, 1 - par

            @pl.when(gg + 1 < NGR)
            def _widx():
                pltpu.make_async_copy(grp_hbm.at[w, gg + 1], idxg.at[nxt],
                                      isems.at[nxt]).wait()

            for b in range(NB):
                pltpu.make_async_copy(y_hbm.at[idxg.at[cur, b]],
                                      rowsb.at[pl.ds(b * CH, CH)],
                                      gsems.at[b]).wait()
                pltpu.sync_copy(rowsb.at[pl.ds(b * CH, CH)],
                                acc_sh.at[idxg.at[cur, NB + b]], add=True)

                @pl.when(gg + 1 < NGR)
                def _fire():
                    pltpu.async_copy(y_hbm.at[idxg.at[nxt, b]],
                                     rowsb.at[pl.ds(b * CH, CH)],
                                     gsems.at[b])

            @pl.when(gg + 2 < NGR)
            def _nidx():
                pltpu.async_copy(grp_hbm.at[w, gg + 2], idxg.at[cur],
                                 isems.at[cur])
        return carry

    lax.fori_loop(0, NGR // 2, grouppair, 0)
    plsc.subcore_barrier()
    pltpu.sync_copy(acc_sh.at[pl.ds(s * RPS, RPS)],
                    out_hbm.at[c, pl.ds(s * RPS, RPS)])


def _tc_xw_body(x_ref, w_ref, o_ref):
    o_ref[...] = jnp.dot(x_ref[...], w_ref[...],
                         preferred_element_type=jnp.float32)


def _tc_xw(x, w):
    bn = 1024
    return pl.pallas_call(
        _tc_xw_body,
        grid=(NPAD // bn,),
        in_specs=[pl.BlockSpec((bn, D), lambda g: (g, 0)),
                  pl.BlockSpec((D, H), lambda g: (0, 0))],
        out_specs=pl.BlockSpec((bn, H), lambda g: (g, 0)),
        out_shape=jax.ShapeDtypeStruct((NPAD, H), jnp.float32),
    )(x, w)


def _tc_scale_body(d0_ref, d1_ref, xw_ref, y_ref, dinv_ref):
    dinv = lax.rsqrt(d0_ref[...] + d1_ref[...] + 1.0)
    dinv_ref[...] = dinv
    y_ref[...] = xw_ref[...] * dinv


def _tc_scale(d0, d1, xw):
    bn = 1024
    return pl.pallas_call(
        _tc_scale_body,
        grid=(NPAD // bn,),
        in_specs=[pl.BlockSpec((bn, 1), lambda g: (g, 0)),
                  pl.BlockSpec((bn, 1), lambda g: (g, 0)),
                  pl.BlockSpec((bn, H), lambda g: (g, 0))],
        out_specs=[pl.BlockSpec((bn, H), lambda g: (g, 0)),
                   pl.BlockSpec((bn, 1), lambda g: (g, 0))],
        out_shape=[jax.ShapeDtypeStruct((NPAD, H), jnp.float32),
                   jax.ShapeDtypeStruct((NPAD, 1), jnp.float32)],
    )(d0, d1, xw)


def _tc_layer_body(a0_ref, a1_ref, y_ref, dinv_ref, b_ref, w_ref, y2_ref):
    dinv = dinv_ref[...]
    h = jnp.maximum((a0_ref[...] + a1_ref[...] + y_ref[...]) * dinv
                    + b_ref[...], 0.0)
    y2_ref[...] = jnp.dot(h, w_ref[...],
                          preferred_element_type=jnp.float32) * dinv


def _tc_layer(a0, a1, y, dinv, b, w):
    bn = 1024
    return pl.pallas_call(
        _tc_layer_body,
        grid=(NPAD // bn,),
        in_specs=[pl.BlockSpec((bn, H), lambda g: (g, 0)),
                  pl.BlockSpec((bn, H), lambda g: (g, 0)),
                  pl.BlockSpec((bn, H), lambda g: (g, 0)),
                  pl.BlockSpec((bn, 1), lambda g: (g, 0)),
                  pl.BlockSpec((1, H), lambda g: (0, 0)),
                  pl.BlockSpec((H, H), lambda g: (0, 0))],
        out_specs=pl.BlockSpec((bn, H), lambda g: (g, 0)),
        out_shape=jax.ShapeDtypeStruct((NPAD, H), jnp.float32),
    )(a0, a1, y, dinv, b, w)


def _tc_head_body(a0_ref, a1_ref, y_ref, dinv_ref, b_ref, bt_ref,
                  wlin_ref, blin_ref, o_ref, sums_ref, cnts_ref):
    g = pl.program_id(0)
    ng = pl.num_programs(0)
    h = jnp.maximum((a0_ref[...] + a1_ref[...] + y_ref[...]) * dinv_ref[...]
                    + b_ref[...], 0.0)
    bn = h.shape[0]
    seg = lax.broadcasted_iota(jnp.int32, (G, bn), 0)
    m = (bt_ref[...] == seg).astype(jnp.float32)

    @pl.when(g == 0)
    def _init():
        sums_ref[...] = jnp.zeros_like(sums_ref)
        cnts_ref[...] = jnp.zeros_like(cnts_ref)

    sums_ref[...] += jnp.dot(m, h, preferred_element_type=jnp.float32)
    cnts_ref[...] += jnp.dot(m, jnp.ones_like(h),
                             preferred_element_type=jnp.float32)

    @pl.when(g == ng - 1)
    def _final():
        pooled = sums_ref[...] / jnp.maximum(cnts_ref[...], 1.0)
        o_ref[...] = jnp.dot(pooled, wlin_ref[...],
                             preferred_element_type=jnp.float32) + blin_ref[...]


def _tc_head(a0, a1, y2, dinv, b2, batch_t, wlin_pad, blin_pad):
    bn = 1024
    return pl.pallas_call(
        _tc_head_body,
        grid=(NPAD // bn,),
        in_specs=[pl.BlockSpec((bn, H), lambda g: (g, 0)),
                  pl.BlockSpec((bn, H), lambda g: (g, 0)),
                  pl.BlockSpec((bn, H), lambda g: (g, 0)),
                  pl.BlockSpec((bn, 1), lambda g: (g, 0)),
                  pl.BlockSpec((1, H), lambda g: (0, 0)),
                  pl.BlockSpec((1, bn), lambda g: (0, g)),
                  pl.BlockSpec((H, H), lambda g: (0, 0)),
                  pl.BlockSpec((1, H), lambda g: (0, 0))],
        out_specs=pl.BlockSpec((G, H), lambda g: (0, 0)),
        out_shape=jax.ShapeDtypeStruct((G, H), jnp.float32),
        scratch_shapes=[pltpu.VMEM((G, H), jnp.float32),
                        pltpu.VMEM((G, H), jnp.float32)],
    )(a0, a1, y2, dinv, b2, batch_t, wlin_pad, blin_pad)


def kernel(x, edge_index, batch, W1, b1, W2, b2, Wlin, blin):
    dst3 = edge_index[1].reshape(NW, NCH, CH)
    npadE = NW * EPWP - E
    srcp = jnp.concatenate([edge_index[0], jnp.zeros((npadE,), jnp.int32)])
    dstp = jnp.concatenate([edge_index[1],
                            jnp.full((npadE,), NPAD, jnp.int32)])
    grp = jnp.stack([srcp.reshape(NW, NGR, NB, CH),
                     dstp.reshape(NW, NGR, NB, CH)],
                    axis=2).reshape(NW, NGR, 2 * NB, CH)
    x_pad = jnp.concatenate(
        [x, jnp.zeros((NPAD - N, D), jnp.float32)], axis=0)
    batch_t = jnp.concatenate(
        [batch, jnp.full((NPAD - N,), G, jnp.int32)]).reshape(1, NPAD)
    wlin_pad = jnp.concatenate(
        [Wlin, jnp.zeros((H, H - C), jnp.float32)], axis=1)
    blin_pad = jnp.concatenate(
        [blin, jnp.zeros((H - C,), jnp.float32)]).reshape(1, H)
    b1r = b1.reshape(1, H)
    b2r = b2.reshape(1, H)

    degp = _sc_degree(dst3)
    d0 = degp[0, :, 0:1]
    d1 = degp[1, :, 0:1]

    xw1 = _tc_xw(x_pad, W1)
    y1, dinv = _tc_scale(d0, d1, xw1)

    agg1 = _sc_aggregate(y1, grp)

    y2 = _tc_layer(agg1[0], agg1[1], y1, dinv, b1r, W2)

    agg2 = _sc_aggregate(y2, grp)

    out = _tc_head(agg2[0], agg2[1], y2, dinv, b2r, batch_t,
                   wlin_pad, blin_pad)
    return out[:, :C]


# speedup vs baseline: 2.5385x; 2.5385x over previous
import functools

import jax
import jax.numpy as jnp
from jax import lax
from jax.experimental import pallas as pl
from jax.experimental.pallas import tpu as pltpu
from jax.experimental.pallas import tpu_sc as plsc

N = 10000
NPAD = 10240
E = 320000
D = 128
H = 128
C = 10
G = 64

NC = 2
NS = 16
NW = NC * NS
EPW = E // NW
CH = 80
NCH = EPW // CH
NB = 2
CH2 = 100
NCH2 = EPW // CH2
RPS = NPAD // NS

_mesh = plsc.VectorSubcoreMesh(core_axis_name="c", subcore_axis_name="s")


@functools.partial(
    pl.kernel,
    out_type=jax.ShapeDtypeStruct((NC, NPAD, 16), jnp.float32),
    mesh=_mesh,
    scratch_types=[
        pltpu.VMEM((NCH, CH), jnp.int32),
        pltpu.VMEM((CH, 16), jnp.float32),
        pltpu.VMEM((16, 16), jnp.float32),
        pltpu.VMEM_SHARED((NPAD, 16), jnp.float32),
    ],
)
def _sc_degree(dst_hbm, out_hbm, idx_d, ones_b, zb, acc_sh):
    c = lax.axis_index("c")
    s = lax.axis_index("s")
    w = s * NC + c
    pltpu.sync_copy(dst_hbm.at[w], idx_d)

    one16 = jnp.full((16,), 1.0, dtype=jnp.float32)
    zero16 = jnp.zeros((16,), dtype=jnp.float32)

    def fill_ones(i, carry):
        ones_b[i, pl.ds(0, 16)] = one16
        return carry

    lax.fori_loop(0, CH, fill_ones, 0)

    def fill_zero(i, carry):
        zb[i, pl.ds(0, 16)] = zero16
        return carry

    lax.fori_loop(0, 16, fill_zero, 0)

    def zero_acc(i, carry):
        pltpu.sync_copy(zb, acc_sh.at[pl.ds(s * RPS + i * 16, 16)])
        return carry

    lax.fori_loop(0, RPS // 16, zero_acc, 0)
    plsc.subcore_barrier()

    def chunk(j, carry):
        pltpu.sync_copy(ones_b, acc_sh.at[idx_d.at[j]], add=True)
        return carry

    lax.fori_loop(0, NCH, chunk, 0)
    plsc.subcore_barrier()
    pltpu.sync_copy(acc_sh.at[pl.ds(s * RPS, RPS)],
                    out_hbm.at[c, pl.ds(s * RPS, RPS)])


@functools.partial(
    pl.kernel,
    out_type=jax.ShapeDtypeStruct((NC, NPAD, H), jnp.float32),
    mesh=_mesh,
    scratch_types=[
        pltpu.VMEM((NCH2, CH2), jnp.int32),
        pltpu.VMEM((NCH2, CH2), jnp.int32),
        pltpu.VMEM((NB * CH2, H), jnp.float32),
        pltpu.SemaphoreType.DMA((NB,)),
        pltpu.SemaphoreType.DMA((NB,)),
        pltpu.VMEM_SHARED((NPAD, H), jnp.float32),
    ],
    compiler_params=pltpu.CompilerParams(use_tc_tiling_on_sc=False),
)
def _sc_aggregate(y_hbm, src_hbm, dst_hbm, out_hbm, idx_s, idx_d, rowsb,
                  gsems, ssems, acc_sh):
    c = lax.axis_index("c")
    s = lax.axis_index("s")
    w = s * NC + c
    pltpu.sync_copy(src_hbm.at[w], idx_s)
    pltpu.sync_copy(dst_hbm.at[w], idx_d)

    zero16 = jnp.zeros((16,), dtype=jnp.float32)

    def fill_zero(i, carry):
        rowsb[i // 8, pl.ds((i % 8) * 16, 16)] = zero16
        return carry

    lax.fori_loop(0, CH2 * (H // 16), fill_zero, 0)

    def zero_acc(i, carry):
        pltpu.sync_copy(rowsb.at[pl.ds(0, 80)],
                        acc_sh.at[pl.ds(s * RPS + i * 80, 80)])
        return carry

    lax.fori_loop(0, RPS // 80, zero_acc, 0)

    plsc.subcore_barrier()

    def group(g, carry):
        gh = [
            pltpu.async_copy(y_hbm.at[idx_s.at[g * NB + b]],
                             rowsb.at[pl.ds(b * CH2, CH2)], gsems.at[b])
            for b in range(NB)
        ]
        sh = []
        for b in range(NB):
            gh[b].wait()
            sh.append(pltpu.async_copy(rowsb.at[pl.ds(b * CH2, CH2)],
                                       acc_sh.at[idx_d.at[g * NB + b]],
                                       ssems.at[b], add=True))
        for b in range(NB):
            sh[b].wait()
        return carry

    lax.fori_loop(0, NCH2 // NB, group, 0)
    plsc.subcore_barrier()
    pltpu.sync_copy(acc_sh.at[pl.ds(s * RPS, RPS)],
                    out_hbm.at[c, pl.ds(s * RPS, RPS)])


def _tc_xw_body(x_ref, w_ref, o_ref):
    o_ref[...] = jnp.dot(x_ref[...], w_ref[...],
                         preferred_element_type=jnp.float32)


def _tc_xw(x, w):
    bn = 1024
    return pl.pallas_call(
        _tc_xw_body,
        grid=(NPAD // bn,),
        in_specs=[pl.BlockSpec((bn, D), lambda g: (g, 0)),
                  pl.BlockSpec((D, H), lambda g: (0, 0))],
        out_specs=pl.BlockSpec((bn, H), lambda g: (g, 0)),
        out_shape=jax.ShapeDtypeStruct((NPAD, H), jnp.float32),
    )(x, w)


def _tc_scale_body(d0_ref, d1_ref, xw_ref, y_ref, dinv_ref):
    dinv = lax.rsqrt(d0_ref[...] + d1_ref[...] + 1.0)
    dinv_ref[...] = dinv
    y_ref[...] = xw_ref[...] * dinv


def _tc_scale(d0, d1, xw):
    bn = 1024
    return pl.pallas_call(
        _tc_scale_body,
        grid=(NPAD // bn,),
        in_specs=[pl.BlockSpec((bn, 1), lambda g: (g, 0)),
                  pl.BlockSpec((bn, 1), lambda g: (g, 0)),
                  pl.BlockSpec((bn, H), lambda g: (g, 0))],
        out_specs=[pl.BlockSpec((bn, H), lambda g: (g, 0)),
                   pl.BlockSpec((bn, 1), lambda g: (g, 0))],
        out_shape=[jax.ShapeDtypeStruct((NPAD, H), jnp.float32),
                   jax.ShapeDtypeStruct((NPAD, 1), jnp.float32)],
    )(d0, d1, xw)


def _tc_layer_body(a0_ref, a1_ref, y_ref, dinv_ref, b_ref, w_ref, y2_ref):
    dinv = dinv_ref[...]
    h = jnp.maximum((a0_ref[...] + a1_ref[...] + y_ref[...]) * dinv
                    + b_ref[...], 0.0)
    y2_ref[...] = jnp.dot(h, w_ref[...],
                          preferred_element_type=jnp.float32) * dinv


def _tc_layer(a0, a1, y, dinv, b, w):
    bn = 1024
    return pl.pallas_call(
        _tc_layer_body,
        grid=(NPAD // bn,),
        in_specs=[pl.BlockSpec((bn, H), lambda g: (g, 0)),
                  pl.BlockSpec((bn, H), lambda g: (g, 0)),
                  pl.BlockSpec((bn, H), lambda g: (g, 0)),
                  pl.BlockSpec((bn, 1), lambda g: (g, 0)),
                  pl.BlockSpec((1, H), lambda g: (0, 0)),
                  pl.BlockSpec((H, H), lambda g: (0, 0))],
        out_specs=pl.BlockSpec((bn, H), lambda g: (g, 0)),
        out_shape=jax.ShapeDtypeStruct((NPAD, H), jnp.float32),
    )(a0, a1, y, dinv, b, w)


def _tc_head_body(a0_ref, a1_ref, y_ref, dinv_ref, b_ref, bt_ref,
                  wlin_ref, blin_ref, o_ref, sums_ref, cnts_ref):
    g = pl.program_id(0)
    ng = pl.num_programs(0)
    h = jnp.maximum((a0_ref[...] + a1_ref[...] + y_ref[...]) * dinv_ref[...]
                    + b_ref[...], 0.0)
    bn = h.shape[0]
    seg = lax.broadcasted_iota(jnp.int32, (G, bn), 0)
    m = (bt_ref[...] == seg).astype(jnp.float32)

    @pl.when(g == 0)
    def _init():
        sums_ref[...] = jnp.zeros_like(sums_ref)
        cnts_ref[...] = jnp.zeros_like(cnts_ref)

    sums_ref[...] += jnp.dot(m, h, preferred_element_type=jnp.float32)
    cnts_ref[...] += jnp.dot(m, jnp.ones_like(h),
                             preferred_element_type=jnp.float32)

    @pl.when(g == ng - 1)
    def _final():
        pooled = sums_ref[...] / jnp.maximum(cnts_ref[...], 1.0)
        o_ref[...] = jnp.dot(pooled, wlin_ref[...],
                             preferred_element_type=jnp.float32) + blin_ref[...]


def _tc_head(a0, a1, y2, dinv, b2, batch_t, wlin_pad, blin_pad):
    bn = 1024
    return pl.pallas_call(
        _tc_head_body,
        grid=(NPAD // bn,),
        in_specs=[pl.BlockSpec((bn, H), lambda g: (g, 0)),
                  pl.BlockSpec((bn, H), lambda g: (g, 0)),
                  pl.BlockSpec((bn, H), lambda g: (g, 0)),
                  pl.BlockSpec((bn, 1), lambda g: (g, 0)),
                  pl.BlockSpec((1, H), lambda g: (0, 0)),
                  pl.BlockSpec((1, bn), lambda g: (0, g)),
                  pl.BlockSpec((H, H), lambda g: (0, 0)),
                  pl.BlockSpec((1, H), lambda g: (0, 0))],
        out_specs=pl.BlockSpec((G, H), lambda g: (0, 0)),
        out_shape=jax.ShapeDtypeStruct((G, H), jnp.float32),
        scratch_shapes=[pltpu.VMEM((G, H), jnp.float32),
                        pltpu.VMEM((G, H), jnp.float32)],
    )(a0, a1, y2, dinv, b2, batch_t, wlin_pad, blin_pad)


def kernel(x, edge_index, batch, W1, b1, W2, b2, Wlin, blin):
    dst3 = edge_index[1].reshape(NW, NCH, CH)
    src32 = edge_index[0].reshape(NW, NCH2, CH2)
    dst32 = edge_index[1].reshape(NW, NCH2, CH2)
    x_pad = jnp.concatenate(
        [x, jnp.zeros((NPAD - N, D), jnp.float32)], axis=0)
    batch_t = jnp.concatenate(
        [batch, jnp.full((NPAD - N,), G, jnp.int32)]).reshape(1, NPAD)
    wlin_pad = jnp.concatenate(
        [Wlin, jnp.zeros((H, H - C), jnp.float32)], axis=1)
    blin_pad = jnp.concatenate(
        [blin, jnp.zeros((H - C,), jnp.float32)]).reshape(1, H)
    b1r = b1.reshape(1, H)
    b2r = b2.reshape(1, H)

    degp = _sc_degree(dst3)
    d0 = degp[0, :, 0:1]
    d1 = degp[1, :, 0:1]

    xw1 = _tc_xw(x_pad, W1)
    y1, dinv = _tc_scale(d0, d1, xw1)

    agg1 = _sc_aggregate(y1, src32, dst32)

    y2 = _tc_layer(agg1[0], agg1[1], y1, dinv, b1r, W2)

    agg2 = _sc_aggregate(y2, src32, dst32)

    out = _tc_head(agg2[0], agg2[1], y2, dinv, b2r, batch_t,
                   wlin_pad, blin_pad)
    return out[:, :C]


# speedup vs baseline: 3.1212x; 1.2296x over previous
import functools

import jax
import jax.numpy as jnp
from jax import lax
from jax.experimental import pallas as pl
from jax.experimental.pallas import tpu as pltpu
from jax.experimental.pallas import tpu_sc as plsc

N = 10000
NPAD = 10240
E = 320000
D = 128
H = 128
C = 10
G = 64

NC = 2
NS = 16
NW = NC * NS
EPW = E // NW
CH = 80
NCH = EPW // CH
NB = 2
CH2 = 50
NCH2 = EPW // CH2
RPS = NPAD // NS

_mesh = plsc.VectorSubcoreMesh(core_axis_name="c", subcore_axis_name="s")


@functools.partial(
    pl.kernel,
    out_type=jax.ShapeDtypeStruct((NC, NPAD, 16), jnp.float32),
    mesh=_mesh,
    scratch_types=[
        pltpu.VMEM((NCH, CH), jnp.int32),
        pltpu.VMEM((CH, 16), jnp.float32),
        pltpu.VMEM((16, 16), jnp.float32),
        pltpu.VMEM_SHARED((NPAD, 16), jnp.float32),
    ],
)
def _sc_degree(dst_hbm, out_hbm, idx_d, ones_b, zb, acc_sh):
    c = lax.axis_index("c")
    s = lax.axis_index("s")
    w = s * NC + c
    pltpu.sync_copy(dst_hbm.at[w], idx_d)

    one16 = jnp.full((16,), 1.0, dtype=jnp.float32)
    zero16 = jnp.zeros((16,), dtype=jnp.float32)

    def fill_ones(i, carry):
        ones_b[i, pl.ds(0, 16)] = one16
        return carry

    lax.fori_loop(0, CH, fill_ones, 0)

    def fill_zero(i, carry):
        zb[i, pl.ds(0, 16)] = zero16
        return carry

    lax.fori_loop(0, 16, fill_zero, 0)

    def zero_acc(i, carry):
        pltpu.sync_copy(zb, acc_sh.at[pl.ds(s * RPS + i * 16, 16)])
        return carry

    lax.fori_loop(0, RPS // 16, zero_acc, 0)
    plsc.subcore_barrier()

    def chunk(j, carry):
        pltpu.sync_copy(ones_b, acc_sh.at[idx_d.at[j]], add=True)
        return carry

    lax.fori_loop(0, NCH, chunk, 0)
    plsc.subcore_barrier()
    pltpu.sync_copy(acc_sh.at[pl.ds(s * RPS, RPS)],
                    out_hbm.at[c, pl.ds(s * RPS, RPS)])


@functools.partial(
    pl.kernel,
    out_type=jax.ShapeDtypeStruct((NC, NPAD, H), jnp.float32),
    mesh=_mesh,
    scratch_types=[
        pltpu.VMEM((NCH2, CH2), jnp.int32),
        pltpu.VMEM((NCH2, CH2), jnp.int32),
        pltpu.VMEM((4 * CH2, H), jnp.float32),
        pltpu.SemaphoreType.DMA((4,)),
        pltpu.SemaphoreType.DMA((4,)),
        pltpu.VMEM_SHARED((NPAD, H), jnp.float32),
    ],
    compiler_params=pltpu.CompilerParams(use_tc_tiling_on_sc=False),
)
def _sc_aggregate(y_hbm, src_hbm, dst_hbm, out_hbm, idx_s, idx_d, rowsb,
                  gsems, ssems, acc_sh):
    c = lax.axis_index("c")
    s = lax.axis_index("s")
    w = s * NC + c
    pltpu.sync_copy(src_hbm.at[w], idx_s)
    pltpu.sync_copy(dst_hbm.at[w], idx_d)

    zero16 = jnp.zeros((16,), dtype=jnp.float32)

    def fill_zero(i, carry):
        rowsb[i // 8, pl.ds((i % 8) * 16, 16)] = zero16
        return carry

    lax.fori_loop(0, 80 * (H // 16), fill_zero, 0)

    def zero_acc(i, carry):
        pltpu.sync_copy(rowsb.at[pl.ds(0, 80)],
                        acc_sh.at[pl.ds(s * RPS + i * 80, 80)])
        return carry

    lax.fori_loop(0, RPS // 80, zero_acc, 0)

    plsc.subcore_barrier()

    NBUF, LAG = 4, 2
    gh = [None] * NCH2
    sh = [None] * NCH2

    def buf(i):
        return rowsb.at[pl.ds((i % NBUF) * CH2, CH2)]

    for i in range(NCH2 + LAG):
        if i < NCH2:
            if i >= NBUF:
                sh[i - NBUF].wait()
            gh[i] = pltpu.async_copy(y_hbm.at[idx_s.at[i]], buf(i),
                                     gsems.at[i % NBUF])
        if i >= LAG:
            j = i - LAG
            gh[j].wait()
            sh[j] = pltpu.async_copy(buf(j), acc_sh.at[idx_d.at[j]],
                                     ssems.at[j % NBUF], add=True)
    for j in range(NCH2 - NBUF, NCH2):
        sh[j].wait()

    plsc.subcore_barrier()
    pltpu.sync_copy(acc_sh.at[pl.ds(s * RPS, RPS)],
                    out_hbm.at[c, pl.ds(s * RPS, RPS)])


def _tc_xw_body(x_ref, w_ref, o_ref):
    o_ref[...] = jnp.dot(x_ref[...], w_ref[...],
                         preferred_element_type=jnp.float32)


def _tc_xw(x, w):
    bn = 1024
    return pl.pallas_call(
        _tc_xw_body,
        grid=(NPAD // bn,),
        in_specs=[pl.BlockSpec((bn, D), lambda g: (g, 0)),
                  pl.BlockSpec((D, H), lambda g: (0, 0))],
        out_specs=pl.BlockSpec((bn, H), lambda g: (g, 0)),
        out_shape=jax.ShapeDtypeStruct((NPAD, H), jnp.float32),
    )(x, w)


def _tc_scale_body(d0_ref, d1_ref, xw_ref, y_ref, dinv_ref):
    dinv = lax.rsqrt(d0_ref[...] + d1_ref[...] + 1.0)
    dinv_ref[...] = dinv
    y_ref[...] = xw_ref[...] * dinv


def _tc_scale(d0, d1, xw):
    bn = 1024
    return pl.pallas_call(
        _tc_scale_body,
        grid=(NPAD // bn,),
        in_specs=[pl.BlockSpec((bn, 1), lambda g: (g, 0)),
                  pl.BlockSpec((bn, 1), lambda g: (g, 0)),
                  pl.BlockSpec((bn, H), lambda g: (g, 0))],
        out_specs=[pl.BlockSpec((bn, H), lambda g: (g, 0)),
                   pl.BlockSpec((bn, 1), lambda g: (g, 0))],
        out_shape=[jax.ShapeDtypeStruct((NPAD, H), jnp.float32),
                   jax.ShapeDtypeStruct((NPAD, 1), jnp.float32)],
    )(d0, d1, xw)


def _tc_layer_body(a0_ref, a1_ref, y_ref, dinv_ref, b_ref, w_ref, y2_ref):
    dinv = dinv_ref[...]
    h = jnp.maximum((a0_ref[...] + a1_ref[...] + y_ref[...]) * dinv
                    + b_ref[...], 0.0)
    y2_ref[...] = jnp.dot(h, w_ref[...],
                          preferred_element_type=jnp.float32) * dinv


def _tc_layer(a0, a1, y, dinv, b, w):
    bn = 1024
    return pl.pallas_call(
        _tc_layer_body,
        grid=(NPAD // bn,),
        in_specs=[pl.BlockSpec((bn, H), lambda g: (g, 0)),
                  pl.BlockSpec((bn, H), lambda g: (g, 0)),
                  pl.BlockSpec((bn, H), lambda g: (g, 0)),
                  pl.BlockSpec((bn, 1), lambda g: (g, 0)),
                  pl.BlockSpec((1, H), lambda g: (0, 0)),
                  pl.BlockSpec((H, H), lambda g: (0, 0))],
        out_specs=pl.BlockSpec((bn, H), lambda g: (g, 0)),
        out_shape=jax.ShapeDtypeStruct((NPAD, H), jnp.float32),
    )(a0, a1, y, dinv, b, w)


def _tc_head_body(a0_ref, a1_ref, y_ref, dinv_ref, b_ref, bt_ref,
                  wlin_ref, blin_ref, o_ref, sums_ref, cnts_ref):
    g = pl.program_id(0)
    ng = pl.num_programs(0)
    h = jnp.maximum((a0_ref[...] + a1_ref[...] + y_ref[...]) * dinv_ref[...]
                    + b_ref[...], 0.0)
    bn = h.shape[0]
    seg = lax.broadcasted_iota(jnp.int32, (G, bn), 0)
    m = (bt_ref[...] == seg).astype(jnp.float32)

    @pl.when(g == 0)
    def _init():
        sums_ref[...] = jnp.zeros_like(sums_ref)
        cnts_ref[...] = jnp.zeros_like(cnts_ref)

    sums_ref[...] += jnp.dot(m, h, preferred_element_type=jnp.float32)
    cnts_ref[...] += jnp.dot(m, jnp.ones_like(h),
                             preferred_element_type=jnp.float32)

    @pl.when(g == ng - 1)
    def _final():
        pooled = sums_ref[...] / jnp.maximum(cnts_ref[...], 1.0)
        o_ref[...] = jnp.dot(pooled, wlin_ref[...],
                             preferred_element_type=jnp.float32) + blin_ref[...]


def _tc_head(a0, a1, y2, dinv, b2, batch_t, wlin_pad, blin_pad):
    bn = 1024
    return pl.pallas_call(
        _tc_head_body,
        grid=(NPAD // bn,),
        in_specs=[pl.BlockSpec((bn, H), lambda g: (g, 0)),
                  pl.BlockSpec((bn, H), lambda g: (g, 0)),
                  pl.BlockSpec((bn, H), lambda g: (g, 0)),
                  pl.BlockSpec((bn, 1), lambda g: (g, 0)),
                  pl.BlockSpec((1, H), lambda g: (0, 0)),
                  pl.BlockSpec((1, bn), lambda g: (0, g)),
                  pl.BlockSpec((H, H), lambda g: (0, 0)),
                  pl.BlockSpec((1, H), lambda g: (0, 0))],
        out_specs=pl.BlockSpec((G, H), lambda g: (0, 0)),
        out_shape=jax.ShapeDtypeStruct((G, H), jnp.float32),
        scratch_shapes=[pltpu.VMEM((G, H), jnp.float32),
                        pltpu.VMEM((G, H), jnp.float32)],
    )(a0, a1, y2, dinv, b2, batch_t, wlin_pad, blin_pad)


def kernel(x, edge_index, batch, W1, b1, W2, b2, Wlin, blin):
    dst3 = edge_index[1].reshape(NW, NCH, CH)
    src32 = edge_index[0].reshape(NW, NCH2, CH2)
    dst32 = edge_index[1].reshape(NW, NCH2, CH2)
    x_pad = jnp.concatenate(
        [x, jnp.zeros((NPAD - N, D), jnp.float32)], axis=0)
    batch_t = jnp.concatenate(
        [batch, jnp.full((NPAD - N,), G, jnp.int32)]).reshape(1, NPAD)
    wlin_pad = jnp.concatenate(
        [Wlin, jnp.zeros((H, H - C), jnp.float32)], axis=1)
    blin_pad = jnp.concatenate(
        [blin, jnp.zeros((H - C,), jnp.float32)]).reshape(1, H)
    b1r = b1.reshape(1, H)
    b2r = b2.reshape(1, H)

    degp = _sc_degree(dst3)
    d0 = degp[0, :, 0:1]
    d1 = degp[1, :, 0:1]

    xw1 = _tc_xw(x_pad, W1)
    y1, dinv = _tc_scale(d0, d1, xw1)

    agg1 = _sc_aggregate(y1, src32, dst32)

    y2 = _tc_layer(agg1[0], agg1[1], y1, dinv, b1r, W2)

    agg2 = _sc_aggregate(y2, src32, dst32)

    out = _tc_head(agg2[0], agg2[1], y2, dinv, b2r, batch_t,
                   wlin_pad, blin_pad)
    return out[:, :C]


# speedup vs baseline: 3.2237x; 1.0328x over previous
import functools

import jax
import jax.numpy as jnp
from jax import lax
from jax.experimental import pallas as pl
from jax.experimental.pallas import tpu as pltpu
from jax.experimental.pallas import tpu_sc as plsc

N = 10000
NPAD = 10240
E = 320000
D = 128
H = 128
C = 10
G = 64

NC = 2
NS = 16
NW = NC * NS
EPW = E // NW
CH = 80
NCH = EPW // CH
NB = 2
CH2 = 50
NCH2 = EPW // CH2
RPS = NPAD // NS

_mesh = plsc.VectorSubcoreMesh(core_axis_name="c", subcore_axis_name="s")


@functools.partial(
    pl.kernel,
    out_type=jax.ShapeDtypeStruct((NC, NPAD, 16), jnp.float32),
    mesh=_mesh,
    scratch_types=[
        pltpu.VMEM((NCH, CH), jnp.int32),
        pltpu.VMEM((CH, 16), jnp.float32),
        pltpu.VMEM((16, 16), jnp.float32),
        pltpu.SemaphoreType.DMA,
        pltpu.VMEM_SHARED((NPAD, 16), jnp.float32),
    ],
)
def _sc_degree(dst_hbm, out_hbm, idx_d, ones_b, zb, dsem, acc_sh):
    c = lax.axis_index("c")
    s = lax.axis_index("s")
    w = s * NC + c
    pltpu.sync_copy(dst_hbm.at[w], idx_d)

    one16 = jnp.full((16,), 1.0, dtype=jnp.float32)
    zero16 = jnp.zeros((16,), dtype=jnp.float32)

    def fill_ones(i, carry):
        ones_b[i, pl.ds(0, 16)] = one16
        return carry

    lax.fori_loop(0, CH, fill_ones, 0)

    def fill_zero(i, carry):
        zb[i, pl.ds(0, 16)] = zero16
        return carry

    lax.fori_loop(0, 16, fill_zero, 0)

    def zero_acc(i, carry):
        pltpu.sync_copy(zb, acc_sh.at[pl.ds(s * RPS + i * 16, 16)])
        return carry

    lax.fori_loop(0, RPS // 16, zero_acc, 0)
    plsc.subcore_barrier()

    hs = [pltpu.async_copy(ones_b, acc_sh.at[idx_d.at[j]], dsem, add=True)
          for j in range(NCH)]
    for h in hs:
        h.wait()
    plsc.subcore_barrier()
    pltpu.sync_copy(acc_sh.at[pl.ds(s * RPS, RPS)],
                    out_hbm.at[c, pl.ds(s * RPS, RPS)])


@functools.partial(
    pl.kernel,
    out_type=jax.ShapeDtypeStruct((NC, NPAD, H), jnp.float32),
    mesh=_mesh,
    scratch_types=[
        pltpu.VMEM((NCH2, CH2), jnp.int32),
        pltpu.VMEM((NCH2, CH2), jnp.int32),
        pltpu.VMEM((4 * CH2, H), jnp.float32),
        pltpu.SemaphoreType.DMA((4,)),
        pltpu.SemaphoreType.DMA((4,)),
        pltpu.VMEM_SHARED((NPAD, H), jnp.float32),
    ],
    compiler_params=pltpu.CompilerParams(use_tc_tiling_on_sc=False),
)
def _sc_aggregate(y_hbm, src_hbm, dst_hbm, out_hbm, idx_s, idx_d, rowsb,
                  gsems, ssems, acc_sh):
    c = lax.axis_index("c")
    s = lax.axis_index("s")
    w = s * NC + c
    pltpu.sync_copy(src_hbm.at[w], idx_s)
    pltpu.sync_copy(dst_hbm.at[w], idx_d)

    zero16 = jnp.zeros((16,), dtype=jnp.float32)

    def fill_zero(i, carry):
        rowsb[i // 8, pl.ds((i % 8) * 16, 16)] = zero16
        return carry

    lax.fori_loop(0, 80 * (H // 16), fill_zero, 0)

    def zero_acc(i, carry):
        pltpu.sync_copy(rowsb.at[pl.ds(0, 80)],
                        acc_sh.at[pl.ds(s * RPS + i * 80, 80)])
        return carry

    lax.fori_loop(0, RPS // 80, zero_acc, 0)

    plsc.subcore_barrier()

    NBUF, LAG = 4, 2
    gh = [None] * NCH2
    sh = [None] * NCH2

    def buf(i):
        return rowsb.at[pl.ds((i % NBUF) * CH2, CH2)]

    for i in range(NCH2 + LAG):
        if i < NCH2:
            if i >= NBUF:
                sh[i - NBUF].wait()
            gh[i] = pltpu.async_copy(y_hbm.at[idx_s.at[i]], buf(i),
                                     gsems.at[i % NBUF])
        if i >= LAG:
            j = i - LAG
            gh[j].wait()
            sh[j] = pltpu.async_copy(buf(j), acc_sh.at[idx_d.at[j]],
                                     ssems.at[j % NBUF], add=True)
    for j in range(NCH2 - NBUF, NCH2):
        sh[j].wait()

    plsc.subcore_barrier()
    pltpu.sync_copy(acc_sh.at[pl.ds(s * RPS, RPS)],
                    out_hbm.at[c, pl.ds(s * RPS, RPS)])


def _tc_scale_body(d0_ref, d1_ref, x_ref, w_ref, y_ref, dinv_ref):
    dinv = lax.rsqrt(d0_ref[...] + d1_ref[...] + 1.0)
    dinv_ref[...] = dinv
    y_ref[...] = jnp.dot(x_ref[...], w_ref[...],
                         preferred_element_type=jnp.float32) * dinv


def _tc_scale(d0, d1, x, w):
    bn = 1024
    return pl.pallas_call(
        _tc_scale_body,
        grid=(NPAD // bn,),
        in_specs=[pl.BlockSpec((bn, 1), lambda g: (g, 0)),
                  pl.BlockSpec((bn, 1), lambda g: (g, 0)),
                  pl.BlockSpec((bn, D), lambda g: (g, 0)),
                  pl.BlockSpec((D, H), lambda g: (0, 0))],
        out_specs=[pl.BlockSpec((bn, H), lambda g: (g, 0)),
                   pl.BlockSpec((bn, 1), lambda g: (g, 0))],
        out_shape=[jax.ShapeDtypeStruct((NPAD, H), jnp.float32),
                   jax.ShapeDtypeStruct((NPAD, 1), jnp.float32)],
    )(d0, d1, x, w)


def _tc_layer_body(a0_ref, a1_ref, y_ref, dinv_ref, b_ref, w_ref, y2_ref):
    dinv = dinv_ref[...]
    h = jnp.maximum((a0_ref[...] + a1_ref[...] + y_ref[...]) * dinv
                    + b_ref[...], 0.0)
    y2_ref[...] = jnp.dot(h, w_ref[...],
                          preferred_element_type=jnp.float32) * dinv


def _tc_layer(a0, a1, y, dinv, b, w):
    bn = 1024
    return pl.pallas_call(
        _tc_layer_body,
        grid=(NPAD // bn,),
        in_specs=[pl.BlockSpec((bn, H), lambda g: (g, 0)),
                  pl.BlockSpec((bn, H), lambda g: (g, 0)),
                  pl.BlockSpec((bn, H), lambda g: (g, 0)),
                  pl.BlockSpec((bn, 1), lambda g: (g, 0)),
                  pl.BlockSpec((1, H), lambda g: (0, 0)),
                  pl.BlockSpec((H, H), lambda g: (0, 0))],
        out_specs=pl.BlockSpec((bn, H), lambda g: (g, 0)),
        out_shape=jax.ShapeDtypeStruct((NPAD, H), jnp.float32),
    )(a0, a1, y, dinv, b, w)


def _tc_head_body(a0_ref, a1_ref, y_ref, dinv_ref, b_ref, bt_ref,
                  wlin_ref, blin_ref, o_ref, sums_ref, cnts_ref):
    g = pl.program_id(0)
    ng = pl.num_programs(0)
    h = jnp.maximum((a0_ref[...] + a1_ref[...] + y_ref[...]) * dinv_ref[...]
                    + b_ref[...], 0.0)
    bn = h.shape[0]
    seg = lax.broadcasted_iota(jnp.int32, (G, bn), 0)
    m = (bt_ref[...] == seg).astype(jnp.float32)

    @pl.when(g == 0)
    def _init():
        sums_ref[...] = jnp.zeros_like(sums_ref)
        cnts_ref[...] = jnp.zeros_like(cnts_ref)

    sums_ref[...] += jnp.dot(m, h, preferred_element_type=jnp.float32)
    cnts_ref[...] += jnp.dot(m, jnp.ones_like(h),
                             preferred_element_type=jnp.float32)

    @pl.when(g == ng - 1)
    def _final():
        pooled = sums_ref[...] / jnp.maximum(cnts_ref[...], 1.0)
        o_ref[...] = jnp.dot(pooled, wlin_ref[...],
                             preferred_element_type=jnp.float32) + blin_ref[...]


def _tc_head(a0, a1, y2, dinv, b2, batch_t, wlin_pad, blin_pad):
    bn = 1024
    return pl.pallas_call(
        _tc_head_body,
        grid=(NPAD // bn,),
        in_specs=[pl.BlockSpec((bn, H), lambda g: (g, 0)),
                  pl.BlockSpec((bn, H), lambda g: (g, 0)),
                  pl.BlockSpec((bn, H), lambda g: (g, 0)),
                  pl.BlockSpec((bn, 1), lambda g: (g, 0)),
                  pl.BlockSpec((1, H), lambda g: (0, 0)),
                  pl.BlockSpec((1, bn), lambda g: (0, g)),
                  pl.BlockSpec((H, H), lambda g: (0, 0)),
                  pl.BlockSpec((1, H), lambda g: (0, 0))],
        out_specs=pl.BlockSpec((G, H), lambda g: (0, 0)),
        out_shape=jax.ShapeDtypeStruct((G, H), jnp.float32),
        scratch_shapes=[pltpu.VMEM((G, H), jnp.float32),
                        pltpu.VMEM((G, H), jnp.float32)],
    )(a0, a1, y2, dinv, b2, batch_t, wlin_pad, blin_pad)


def kernel(x, edge_index, batch, W1, b1, W2, b2, Wlin, blin):
    dst3 = edge_index[1].reshape(NW, NCH, CH)
    src32 = edge_index[0].reshape(NW, NCH2, CH2)
    dst32 = edge_index[1].reshape(NW, NCH2, CH2)
    x_pad = jnp.concatenate(
        [x, jnp.zeros((NPAD - N, D), jnp.float32)], axis=0)
    batch_t = jnp.concatenate(
        [batch, jnp.full((NPAD - N,), G, jnp.int32)]).reshape(1, NPAD)
    wlin_pad = jnp.concatenate(
        [Wlin, jnp.zeros((H, H - C), jnp.float32)], axis=1)
    blin_pad = jnp.concatenate(
        [blin, jnp.zeros((H - C,), jnp.float32)]).reshape(1, H)
    b1r = b1.reshape(1, H)
    b2r = b2.reshape(1, H)

    degp = _sc_degree(dst3)
    d0 = degp[0, :, 0:1]
    d1 = degp[1, :, 0:1]

    y1, dinv = _tc_scale(d0, d1, x_pad, W1)

    agg1 = _sc_aggregate(y1, src32, dst32)

    y2 = _tc_layer(agg1[0], agg1[1], y1, dinv, b1r, W2)

    agg2 = _sc_aggregate(y2, src32, dst32)

    out = _tc_head(agg2[0], agg2[1], y2, dinv, b2r, batch_t,
                   wlin_pad, blin_pad)
    return out[:, :C]


# speedup vs baseline: 3.3917x; 1.0521x over previous
import functools

import jax
import jax.numpy as jnp
from jax import lax
from jax.experimental import pallas as pl
from jax.experimental.pallas import tpu as pltpu
from jax.experimental.pallas import tpu_sc as plsc

N = 10000
NPAD = 10240
E = 320000
D = 128
H = 128
C = 10
G = 64

NC = 2
NS = 16
NW = NC * NS
EPW = E // NW
CH = 80
NCH = EPW // CH
NB = 2
CH2 = 40
NCH2 = EPW // CH2
RPS = NPAD // NS

_mesh = plsc.VectorSubcoreMesh(core_axis_name="c", subcore_axis_name="s")


@functools.partial(
    pl.kernel,
    out_type=jax.ShapeDtypeStruct((NC, NPAD, 16), jnp.float32),
    mesh=_mesh,
    scratch_types=[
        pltpu.VMEM((NCH, CH), jnp.int32),
        pltpu.VMEM((CH, 16), jnp.float32),
        pltpu.VMEM((16, 16), jnp.float32),
        pltpu.SemaphoreType.DMA,
        pltpu.VMEM_SHARED((NPAD, 16), jnp.float32),
    ],
)
def _sc_degree(dst_hbm, out_hbm, idx_d, ones_b, zb, dsem, acc_sh):
    c = lax.axis_index("c")
    s = lax.axis_index("s")
    w = s * NC + c
    pltpu.sync_copy(dst_hbm.at[w], idx_d)

    one16 = jnp.full((16,), 1.0, dtype=jnp.float32)
    zero16 = jnp.zeros((16,), dtype=jnp.float32)

    def fill_ones(i, carry):
        ones_b[i, pl.ds(0, 16)] = one16
        return carry

    lax.fori_loop(0, CH, fill_ones, 0)

    def fill_zero(i, carry):
        zb[i, pl.ds(0, 16)] = zero16
        return carry

    lax.fori_loop(0, 16, fill_zero, 0)

    def zero_acc(i, carry):
        pltpu.sync_copy(zb, acc_sh.at[pl.ds(s * RPS + i * 16, 16)])
        return carry

    lax.fori_loop(0, RPS // 16, zero_acc, 0)
    plsc.subcore_barrier()

    hs = [pltpu.async_copy(ones_b, acc_sh.at[idx_d.at[j]], dsem, add=True)
          for j in range(NCH)]
    for h in hs:
        h.wait()
    plsc.subcore_barrier()
    pltpu.sync_copy(acc_sh.at[pl.ds(s * RPS, RPS)],
                    out_hbm.at[c, pl.ds(s * RPS, RPS)])


@functools.partial(
    pl.kernel,
    out_type=jax.ShapeDtypeStruct((NC, NPAD, H), jnp.float32),
    mesh=_mesh,
    scratch_types=[
        pltpu.VMEM((NCH2, CH2), jnp.int32),
        pltpu.VMEM((NCH2, CH2), jnp.int32),
        pltpu.VMEM((5 * CH2, H), jnp.float32),
        pltpu.SemaphoreType.DMA((5,)),
        pltpu.SemaphoreType.DMA((5,)),
        pltpu.VMEM_SHARED((NPAD, H), jnp.float32),
    ],
    compiler_params=pltpu.CompilerParams(use_tc_tiling_on_sc=False),
)
def _sc_aggregate(y_hbm, src_hbm, dst_hbm, out_hbm, idx_s, idx_d, rowsb,
                  gsems, ssems, acc_sh):
    c = lax.axis_index("c")
    s = lax.axis_index("s")
    w = s * NC + c
    pltpu.sync_copy(src_hbm.at[w], idx_s)
    pltpu.sync_copy(dst_hbm.at[w], idx_d)

    zero16 = jnp.zeros((16,), dtype=jnp.float32)

    def fill_zero(i, carry):
        rowsb[i // 8, pl.ds((i % 8) * 16, 16)] = zero16
        return carry

    lax.fori_loop(0, 80 * (H // 16), fill_zero, 0)

    def zero_acc(i, carry):
        pltpu.sync_copy(rowsb.at[pl.ds(0, 80)],
                        acc_sh.at[pl.ds(s * RPS + i * 80, 80)])
        return carry

    lax.fori_loop(0, RPS // 80, zero_acc, 0)

    plsc.subcore_barrier()

    NBUF, LAG = 5, 3
    gh = [None] * NCH2
    sh = [None] * NCH2

    def buf(i):
        return rowsb.at[pl.ds((i % NBUF) * CH2, CH2)]

    for i in range(NCH2 + LAG):
        if i < NCH2:
            if i >= NBUF:
                sh[i - NBUF].wait()
            gh[i] = pltpu.async_copy(y_hbm.at[idx_s.at[i]], buf(i),
                                     gsems.at[i % NBUF])
        if i >= LAG:
            j = i - LAG
            gh[j].wait()
            sh[j] = pltpu.async_copy(buf(j), acc_sh.at[idx_d.at[j]],
                                     ssems.at[j % NBUF], add=True)
    for j in range(NCH2 - NBUF, NCH2):
        sh[j].wait()

    plsc.subcore_barrier()
    pltpu.sync_copy(acc_sh.at[pl.ds(s * RPS, RPS)],
                    out_hbm.at[c, pl.ds(s * RPS, RPS)])


def _tc_scale_body(d0_ref, d1_ref, x_ref, w_ref, y_ref, dinv_ref):
    dinv = lax.rsqrt(d0_ref[...] + d1_ref[...] + 1.0)
    dinv_ref[...] = dinv
    y_ref[...] = jnp.dot(x_ref[...], w_ref[...],
                         preferred_element_type=jnp.float32) * dinv


def _tc_scale(d0, d1, x, w):
    bn = 1024
    return pl.pallas_call(
        _tc_scale_body,
        grid=(NPAD // bn,),
        in_specs=[pl.BlockSpec((bn, 1), lambda g: (g, 0)),
                  pl.BlockSpec((bn, 1), lambda g: (g, 0)),
                  pl.BlockSpec((bn, D), lambda g: (g, 0)),
                  pl.BlockSpec((D, H), lambda g: (0, 0))],
        out_specs=[pl.BlockSpec((bn, H), lambda g: (g, 0)),
                   pl.BlockSpec((bn, 1), lambda g: (g, 0))],
        out_shape=[jax.ShapeDtypeStruct((NPAD, H), jnp.float32),
                   jax.ShapeDtypeStruct((NPAD, 1), jnp.float32)],
    )(d0, d1, x, w)


def _tc_layer_body(a0_ref, a1_ref, y_ref, dinv_ref, b_ref, w_ref, y2_ref):
    dinv = dinv_ref[...]
    h = jnp.maximum((a0_ref[...] + a1_ref[...] + y_ref[...]) * dinv
                    + b_ref[...], 0.0)
    y2_ref[...] = jnp.dot(h, w_ref[...],
                          preferred_element_type=jnp.float32) * dinv


def _tc_layer(a0, a1, y, dinv, b, w):
    bn = 1024
    return pl.pallas_call(
        _tc_layer_body,
        grid=(NPAD // bn,),
        in_specs=[pl.BlockSpec((bn, H), lambda g: (g, 0)),
                  pl.BlockSpec((bn, H), lambda g: (g, 0)),
                  pl.BlockSpec((bn, H), lambda g: (g, 0)),
                  pl.BlockSpec((bn, 1), lambda g: (g, 0)),
                  pl.BlockSpec((1, H), lambda g: (0, 0)),
                  pl.BlockSpec((H, H), lambda g: (0, 0))],
        out_specs=pl.BlockSpec((bn, H), lambda g: (g, 0)),
        out_shape=jax.ShapeDtypeStruct((NPAD, H), jnp.float32),
    )(a0, a1, y, dinv, b, w)


def _tc_head_body(a0_ref, a1_ref, y_ref, dinv_ref, b_ref, bt_ref,
                  wlin_ref, blin_ref, o_ref, sums_ref, cnts_ref):
    g = pl.program_id(0)
    ng = pl.num_programs(0)
    h = jnp.maximum((a0_ref[...] + a1_ref[...] + y_ref[...]) * dinv_ref[...]
                    + b_ref[...], 0.0)
    bn = h.shape[0]
    seg = lax.broadcasted_iota(jnp.int32, (G, bn), 0)
    m = (bt_ref[...] == seg).astype(jnp.float32)

    @pl.when(g == 0)
    def _init():
        sums_ref[...] = jnp.zeros_like(sums_ref)
        cnts_ref[...] = jnp.zeros_like(cnts_ref)

    sums_ref[...] += jnp.dot(m, h, preferred_element_type=jnp.float32)
    cnts_ref[...] += jnp.dot(m, jnp.ones_like(h),
                             preferred_element_type=jnp.float32)

    @pl.when(g == ng - 1)
    def _final():
        pooled = sums_ref[...] / jnp.maximum(cnts_ref[...], 1.0)
        o_ref[...] = jnp.dot(pooled, wlin_ref[...],
                             preferred_element_type=jnp.float32) + blin_ref[...]


def _tc_head(a0, a1, y2, dinv, b2, batch_t, wlin_pad, blin_pad):
    bn = 1024
    return pl.pallas_call(
        _tc_head_body,
        grid=(NPAD // bn,),
        in_specs=[pl.BlockSpec((bn, H), lambda g: (g, 0)),
                  pl.BlockSpec((bn, H), lambda g: (g, 0)),
                  pl.BlockSpec((bn, H), lambda g: (g, 0)),
                  pl.BlockSpec((bn, 1), lambda g: (g, 0)),
                  pl.BlockSpec((1, H), lambda g: (0, 0)),
                  pl.BlockSpec((1, bn), lambda g: (0, g)),
                  pl.BlockSpec((H, H), lambda g: (0, 0)),
                  pl.BlockSpec((1, H), lambda g: (0, 0))],
        out_specs=pl.BlockSpec((G, H), lambda g: (0, 0)),
        out_shape=jax.ShapeDtypeStruct((G, H), jnp.float32),
        scratch_shapes=[pltpu.VMEM((G, H), jnp.float32),
                        pltpu.VMEM((G, H), jnp.float32)],
    )(a0, a1, y2, dinv, b2, batch_t, wlin_pad, blin_pad)


def kernel(x, edge_index, batch, W1, b1, W2, b2, Wlin, blin):
    dst3 = edge_index[1].reshape(NW, NCH, CH)
    src32 = edge_index[0].reshape(NW, NCH2, CH2)
    dst32 = edge_index[1].reshape(NW, NCH2, CH2)
    x_pad = jnp.concatenate(
        [x, jnp.zeros((NPAD - N, D), jnp.float32)], axis=0)
    batch_t = jnp.concatenate(
        [batch, jnp.full((NPAD - N,), G, jnp.int32)]).reshape(1, NPAD)
    wlin_pad = jnp.concatenate(
        [Wlin, jnp.zeros((H, H - C), jnp.float32)], axis=1)
    blin_pad = jnp.concatenate(
        [blin, jnp.zeros((H - C,), jnp.float32)]).reshape(1, H)
    b1r = b1.reshape(1, H)
    b2r = b2.reshape(1, H)

    degp = _sc_degree(dst3)
    d0 = degp[0, :, 0:1]
    d1 = degp[1, :, 0:1]

    y1, dinv = _tc_scale(d0, d1, x_pad, W1)

    agg1 = _sc_aggregate(y1, src32, dst32)

    y2 = _tc_layer(agg1[0], agg1[1], y1, dinv, b1r, W2)

    agg2 = _sc_aggregate(y2, src32, dst32)

    out = _tc_head(agg2[0], agg2[1], y2, dinv, b2r, batch_t,
                   wlin_pad, blin_pad)
    return out[:, :C]
